# Initial kernel scaffold; baseline (speedup 1.0000x reference)
import jax
import jax.numpy as jnp
from jax.experimental import pallas as pl


def kernel(src_tokens, embed_table):
    raise NotImplementedError("write your pallas kernel here")


# speedup vs baseline: 1.4599x; 1.4599x over previous
import functools

import numpy as np
import jax
import jax.numpy as jnp
from jax import lax
from jax.experimental import pallas as pl
from jax.experimental.pallas import tpu as pltpu
from jax.experimental.pallas import tpu_sc as plsc

D_MODEL = 128
SEQ_LEN = 200
PAD_IDX = 1
SCALE = float(np.sqrt(np.float32(D_MODEL)))

NUM_CORES = 2
NUM_SUBCORES = 16
NW = NUM_CORES * NUM_SUBCORES
CHUNK = 100
LANES = 16


def _pos_encoding(length, d_model):
    pos = np.arange(length, dtype=np.float32)[:, None]
    div = np.exp(
        np.arange(0, d_model, 2, dtype=np.float32) * (-np.log(10000.0) / d_model)
    )
    pe = np.zeros((length, d_model), dtype=np.float32)
    pe[:, 0::2] = np.sin(pos * div)
    pe[:, 1::2] = np.cos(pos * div)
    return pe


_PE = _pos_encoding(SEQ_LEN, D_MODEL)


def _build(total_tokens):
    per_w = total_tokens // NW
    n_chunks = per_w // CHUNK
    mesh = plsc.VectorSubcoreMesh(core_axis_name="c", subcore_axis_name="s")

    def body(table_h, idx3_h, idxf_h, pe_h, out_h, mask_h,
             idx_v, idxf_v, pe_v, rows_v, mask_v, sem):
        wid = lax.axis_index("s") * NUM_CORES + lax.axis_index("c")
        pltpu.sync_copy(idx3_h.at[wid], idx_v)
        pltpu.sync_copy(idxf_h.at[wid, 0], idxf_v)
        pltpu.sync_copy(pe_h, pe_v)

        def mask_body(k, carry):
            sl = pl.ds(k * LANES, LANES)
            tok = idxf_v[sl]
            mask_v[sl] = jnp.where(tok == PAD_IDX, 1, 0).astype(jnp.int32)
            return carry

        lax.fori_loop(0, per_w // LANES, mask_body, 0)
        pltpu.sync_copy(mask_v, mask_h.at[wid, 0])

        base = wid * n_chunks

        def chunk_body(j, carry):
            pltpu.async_copy(table_h.at[idx_v.at[j]], rows_v, sem).wait()
            pe_off = (j % 2) * CHUNK

            def tok_body(t, c2):
                p = pe_off + t
                for d in range(D_MODEL // LANES):
                    sl = pl.ds(d * LANES, LANES)
                    rows_v[t, sl] = rows_v[t, sl] * SCALE + pe_v[p, sl]
                return c2

            lax.fori_loop(0, CHUNK, tok_body, 0)
            pltpu.sync_copy(rows_v, out_h.at[base + j])
            return carry

        lax.fori_loop(0, n_chunks, chunk_body, 0)

    kfn = pl.kernel(
        body,
        out_type=(
            jax.ShapeDtypeStruct(
                (total_tokens // CHUNK, CHUNK, D_MODEL), jnp.float32),
            jax.ShapeDtypeStruct((NW, 1, per_w), jnp.int32),
        ),
        mesh=mesh,
        scratch_types=[
            pltpu.VMEM((n_chunks, CHUNK), jnp.int32),
            pltpu.VMEM((per_w,), jnp.int32),
            pltpu.VMEM((SEQ_LEN, D_MODEL), jnp.float32),
            pltpu.VMEM((CHUNK, D_MODEL), jnp.float32),
            pltpu.VMEM((per_w,), jnp.int32),
            pltpu.SemaphoreType.DMA,
        ],
    )
    return kfn


def kernel(src_tokens, embed_table):
    B, L = src_tokens.shape
    total = B * L
    tok = src_tokens.astype(jnp.int32)
    idxf = tok.reshape(NW, 1, total // NW)
    idx3 = idxf.reshape(NW, total // NW // CHUNK, CHUNK)
    pe = jnp.asarray(_PE)
    out_chunks, mask_i = _build(total)(embed_table, idx3, idxf, pe)
    out = out_chunks.reshape(B, L, D_MODEL)
    mask = mask_i.reshape(B, L) != 0
    return (out, mask)


# speedup vs baseline: 1.7621x; 1.2070x over previous
import functools

import numpy as np
import jax
import jax.numpy as jnp
from jax import lax
from jax.experimental import pallas as pl
from jax.experimental.pallas import tpu as pltpu
from jax.experimental.pallas import tpu_sc as plsc

D_MODEL = 128
SEQ_LEN = 200
PAD_IDX = 1
SCALE = float(np.sqrt(np.float32(D_MODEL)))

NUM_CORES = 2
NUM_SUBCORES = 16
NW = NUM_CORES * NUM_SUBCORES
CHUNK = 100
LANES = 16
NSLC = D_MODEL // LANES


def _pos_encoding(length, d_model):
    pos = np.arange(length, dtype=np.float32)[:, None]
    div = np.exp(
        np.arange(0, d_model, 2, dtype=np.float32) * (-np.log(10000.0) / d_model)
    )
    pe = np.zeros((length, d_model), dtype=np.float32)
    pe[:, 0::2] = np.sin(pos * div)
    pe[:, 1::2] = np.cos(pos * div)
    return pe


_PE = _pos_encoding(SEQ_LEN, D_MODEL)


def _build(total_tokens):
    per_w = total_tokens // NW
    n_chunks = per_w // CHUNK
    n_groups = n_chunks // 2
    mesh = plsc.VectorSubcoreMesh(core_axis_name="c", subcore_axis_name="s")

    def body(table_h, idx3_h, idxf_h, pe_h, out_h, mask_h,
             idx_v, idxf_v, pe_v, gbuf0, gbuf1, obuf0, obuf1, mask_v,
             gsem0, gsem1, ssem0, ssem1):
        gbuf = (gbuf0, gbuf1)
        obuf = (obuf0, obuf1)
        gsem = (gsem0, gsem1)
        ssem = (ssem0, ssem1)
        wid = lax.axis_index("s") * NUM_CORES + lax.axis_index("c")
        pltpu.sync_copy(idx3_h.at[wid], idx_v)
        pltpu.sync_copy(idxf_h.at[wid, 0], idxf_v)
        pltpu.sync_copy(pe_h, pe_v)
        base = wid * n_chunks

        def gstart(b, j):
            pltpu.make_async_copy(
                table_h.at[idx_v.at[j]], gbuf[b], gsem[b]).start()

        def gwait(b, j):
            pltpu.make_async_copy(
                table_h.at[idx_v.at[j]], gbuf[b], gsem[b]).wait()

        def sstart(b, j):
            pltpu.make_async_copy(obuf[b], out_h.at[base + j], ssem[b]).start()

        def swait(b, j):
            pltpu.make_async_copy(obuf[b], out_h.at[base + j], ssem[b]).wait()

        def compute(b):
            src, dst = gbuf[b], obuf[b]
            poff = b * CHUNK

            def tok_body(t, carry):
                for d in range(NSLC):
                    sl = pl.ds(d * LANES, LANES)
                    dst[t, sl] = src[t, sl] * SCALE + pe_v[poff + t, sl]
                return carry

            lax.fori_loop(0, CHUNK, tok_body, 0, unroll=2)

        def mask_body(k, carry):
            sl = pl.ds(k * LANES, LANES)
            tok = idxf_v[sl]
            mask_v[sl] = jnp.where(tok == PAD_IDX, 1, 0).astype(jnp.int32)
            return carry

        gstart(0, 0)
        gstart(1, 1)

        lax.fori_loop(0, per_w // LANES, mask_body, 0, unroll=4)
        pltpu.sync_copy(mask_v, mask_h.at[wid, 0])

        for b in range(2):
            gwait(b, b)
            compute(b)
            gstart(b, b + 2)
            sstart(b, b)

        def group(g, carry):
            for b in range(2):
                j = 2 * g + b
                gwait(b, j)
                swait(b, j - 2)
                compute(b)

                @pl.when(j + 2 < n_chunks)
                def _():
                    gstart(b, j + 2)

                sstart(b, j)
            return carry

        lax.fori_loop(1, n_groups, group, 0)

        swait(0, n_chunks - 2)
        swait(1, n_chunks - 1)

    kfn = pl.kernel(
        body,
        out_type=(
            jax.ShapeDtypeStruct(
                (total_tokens // CHUNK, CHUNK, D_MODEL), jnp.float32),
            jax.ShapeDtypeStruct((NW, 1, per_w), jnp.int32),
        ),
        mesh=mesh,
        scratch_types=[
            pltpu.VMEM((n_chunks, CHUNK), jnp.int32),
            pltpu.VMEM((per_w,), jnp.int32),
            pltpu.VMEM((SEQ_LEN, D_MODEL), jnp.float32),
            pltpu.VMEM((CHUNK, D_MODEL), jnp.float32),
            pltpu.VMEM((CHUNK, D_MODEL), jnp.float32),
            pltpu.VMEM((CHUNK, D_MODEL), jnp.float32),
            pltpu.VMEM((CHUNK, D_MODEL), jnp.float32),
            pltpu.VMEM((per_w,), jnp.int32),
            pltpu.SemaphoreType.DMA,
            pltpu.SemaphoreType.DMA,
            pltpu.SemaphoreType.DMA,
            pltpu.SemaphoreType.DMA,
        ],
    )
    return kfn


def kernel(src_tokens, embed_table):
    B, L = src_tokens.shape
    total = B * L
    tok = src_tokens.astype(jnp.int32)
    idxf = tok.reshape(NW, 1, total // NW)
    idx3 = idxf.reshape(NW, total // NW // CHUNK, CHUNK)
    pe = jnp.asarray(_PE)
    out_chunks, mask_i = _build(total)(embed_table, idx3, idxf, pe)
    out = out_chunks.reshape(B, L, D_MODEL)
    mask = mask_i.reshape(B, L) != 0
    return (out, mask)
